# baseline (speedup 1.0000x reference)
import functools

import jax
import jax.numpy as jnp
from jax import lax
from jax.experimental import pallas as pl
from jax.experimental.pallas import tpu as pltpu
from jax.experimental.pallas import tpu_sc as plsc

BATCH = 16384
EMB_DIM = 32
LANES = 16
NUM_CORES = 2
NUM_SUBCORES = 16
NUM_WORKERS = NUM_CORES * NUM_SUBCORES
BPW = BATCH // NUM_WORKERS
IDX_CHUNK = 128
NCHUNK = BPW // IDX_CHUNK
SROWS = 250000
PASS = 256
NPASS = BPW // PASS
CPP = PASS // IDX_CHUNK


def _make_kernel():
    mesh = plsc.VectorSubcoreMesh(core_axis_name="c", subcore_axis_name="s")

    @functools.partial(
        pl.kernel,
        out_type=jax.ShapeDtypeStruct((BATCH,), jnp.float32),
        mesh=mesh,
        compiler_params=pltpu.CompilerParams(
            needs_layout_passes=False, use_tc_tiling_on_sc=False),
        scratch_types=[
            pltpu.VMEM((NCHUNK, IDX_CHUNK), jnp.int32),
            pltpu.VMEM((NCHUNK, IDX_CHUNK), jnp.int32),
            pltpu.VMEM((NCHUNK, IDX_CHUNK), jnp.int32),
            pltpu.VMEM((NCHUNK, IDX_CHUNK), jnp.int32),
            pltpu.VMEM((PASS, 128), jnp.float32),
            pltpu.VMEM((PASS, 128), jnp.float32),
            pltpu.VMEM((BPW,), jnp.float32),
            pltpu.SemaphoreType.DMA,
            pltpu.SemaphoreType.DMA,
            pltpu.SemaphoreType.DMA,
        ],
    )
    def cmf_kernel(users_hbm, items_hbm, uemb_hbm, iemb_hbm, out_hbm,
                   usr_v, isr_v, uoff_v, ioff_v, urows_v, irows_v, outv,
                   stsem, usem, isem):
        wid = lax.axis_index("s") * NUM_CORES + lax.axis_index("c")
        base = wid * BPW

        idx_copies = []
        for j in range(NCHUNK):
            idx_copies.append(pltpu.async_copy(
                users_hbm.at[pl.ds(base + j * IDX_CHUNK, IDX_CHUNK)],
                usr_v.at[j], stsem))
            idx_copies.append(pltpu.async_copy(
                items_hbm.at[pl.ds(base + j * IDX_CHUNK, IDX_CHUNK)],
                isr_v.at[j], stsem))
        for cp in idx_copies:
            cp.wait()
        for j in range(NCHUNK):
            for k in range(IDX_CHUNK // LANES):
                sl = pl.ds(k * LANES, LANES)
                ur = usr_v[j, sl]
                uoff_v[j, sl] = (ur & 3) * EMB_DIM
                usr_v[j, sl] = lax.shift_right_logical(ur, 2)
                ir = isr_v[j, sl]
                ioff_v[j, sl] = (ir & 3) * EMB_DIM
                isr_v[j, sl] = lax.shift_right_logical(ir, 2)

        def do_pass(p):
            gathers = []
            for c in range(CPP):
                j = p * CPP + c
                gathers.append(pltpu.async_copy(
                    uemb_hbm.at[usr_v.at[j]],
                    urows_v.at[pl.ds(c * IDX_CHUNK, IDX_CHUNK)], usem))
                gathers.append(pltpu.async_copy(
                    iemb_hbm.at[isr_v.at[j]],
                    irows_v.at[pl.ds(c * IDX_CHUNK, IDX_CHUNK)], isem))
            for cp in gathers:
                cp.wait()

            def group(g, carry):
                rows = g * LANES + lax.iota(jnp.int32, LANES)
                joff = g // (IDX_CHUNK // LANES)
                koff = g % (IDX_CHUNK // LANES)
                sl = pl.ds(koff * LANES, LANES)
                uo = uoff_v[p * CPP + joff, sl]
                io = ioff_v[p * CPP + joff, sl]
                accs = [jnp.zeros((LANES,), jnp.float32) for _ in range(4)]
                for d in range(EMB_DIM):
                    u = plsc.load_gather(urows_v, [rows, uo + d])
                    v = plsc.load_gather(irows_v, [rows, io + d])
                    accs[d % 4] = accs[d % 4] + u * v
                s = (accs[0] + accs[1]) + (accs[2] + accs[3])
                sig = 1.0 / (1.0 + jnp.exp(-s))
                outv[pl.ds(p * PASS + g * LANES, LANES)] = sig
                return carry

            for g in range(PASS // LANES):
                group(g, 0)

        for p in range(NPASS):
            do_pass(p)

        pltpu.sync_copy(outv, out_hbm.at[pl.ds(base, BPW)])

    return cmf_kernel


_cmf = _make_kernel()


def kernel(users, items, user_emb, item_emb):
    uemb2 = user_emb.reshape(SROWS, 128)
    iemb2 = item_emb.reshape(SROWS, 128)
    return _cmf(users, items, uemb2, iemb2)


# speedup vs baseline: 1.0015x; 1.0015x over previous
import functools

import jax
import jax.numpy as jnp
from jax import lax
from jax.experimental import pallas as pl
from jax.experimental.pallas import tpu as pltpu
from jax.experimental.pallas import tpu_sc as plsc

BATCH = 16384
EMB_DIM = 32
LANES = 16
NUM_CORES = 2
NUM_SUBCORES = 16
NUM_WORKERS = NUM_CORES * NUM_SUBCORES
BPW = BATCH // NUM_WORKERS
IDX_CHUNK = 128
NCHUNK = BPW // IDX_CHUNK
SROWS = 250000
PASS = 256
NPASS = BPW // PASS
CPP = PASS // IDX_CHUNK


def _make_kernel():
    mesh = plsc.VectorSubcoreMesh(core_axis_name="c", subcore_axis_name="s")

    @functools.partial(
        pl.kernel,
        out_type=jax.ShapeDtypeStruct((BATCH,), jnp.float32),
        mesh=mesh,
        compiler_params=pltpu.CompilerParams(
            needs_layout_passes=False, use_tc_tiling_on_sc=True),
        scratch_types=[
            pltpu.VMEM((NCHUNK, IDX_CHUNK), jnp.int32),
            pltpu.VMEM((NCHUNK, IDX_CHUNK), jnp.int32),
            pltpu.VMEM((NCHUNK, IDX_CHUNK), jnp.int32),
            pltpu.VMEM((NCHUNK, IDX_CHUNK), jnp.int32),
            pltpu.VMEM((PASS, 128), jnp.float32),
            pltpu.VMEM((PASS, 128), jnp.float32),
            pltpu.VMEM((BPW,), jnp.float32),
            pltpu.SemaphoreType.DMA,
            pltpu.SemaphoreType.DMA,
            pltpu.SemaphoreType.DMA,
        ],
    )
    def cmf_kernel(users_hbm, items_hbm, uemb_hbm, iemb_hbm, out_hbm,
                   usr_v, isr_v, uoff_v, ioff_v, urows_v, irows_v, outv,
                   stsem, usem, isem):
        wid = lax.axis_index("s") * NUM_CORES + lax.axis_index("c")
        base = wid * BPW

        idx_copies = []
        for j in range(NCHUNK):
            idx_copies.append(pltpu.async_copy(
                users_hbm.at[pl.ds(base + j * IDX_CHUNK, IDX_CHUNK)],
                usr_v.at[j], stsem))
            idx_copies.append(pltpu.async_copy(
                items_hbm.at[pl.ds(base + j * IDX_CHUNK, IDX_CHUNK)],
                isr_v.at[j], stsem))
        for cp in idx_copies:
            cp.wait()
        for j in range(NCHUNK):
            for k in range(IDX_CHUNK // LANES):
                sl = pl.ds(k * LANES, LANES)
                ur = usr_v[j, sl]
                uoff_v[j, sl] = (ur & 3) * EMB_DIM
                usr_v[j, sl] = lax.shift_right_logical(ur, 2)
                ir = isr_v[j, sl]
                ioff_v[j, sl] = (ir & 3) * EMB_DIM
                isr_v[j, sl] = lax.shift_right_logical(ir, 2)

        def do_pass(p):
            gathers = []
            for c in range(CPP):
                j = p * CPP + c
                gathers.append(pltpu.async_copy(
                    uemb_hbm.at[usr_v.at[j]],
                    urows_v.at[pl.ds(c * IDX_CHUNK, IDX_CHUNK)], usem))
                gathers.append(pltpu.async_copy(
                    iemb_hbm.at[isr_v.at[j]],
                    irows_v.at[pl.ds(c * IDX_CHUNK, IDX_CHUNK)], isem))
            for cp in gathers:
                cp.wait()

            def group(g, carry):
                rows = g * LANES + lax.iota(jnp.int32, LANES)
                joff = g // (IDX_CHUNK // LANES)
                koff = g % (IDX_CHUNK // LANES)
                sl = pl.ds(koff * LANES, LANES)
                uo = uoff_v[p * CPP + joff, sl]
                io = ioff_v[p * CPP + joff, sl]
                accs = [jnp.zeros((LANES,), jnp.float32) for _ in range(4)]
                for d in range(EMB_DIM):
                    u = plsc.load_gather(urows_v, [rows, uo + d])
                    v = plsc.load_gather(irows_v, [rows, io + d])
                    accs[d % 4] = accs[d % 4] + u * v
                s = (accs[0] + accs[1]) + (accs[2] + accs[3])
                sig = 1.0 / (1.0 + jnp.exp(-s))
                outv[pl.ds(p * PASS + g * LANES, LANES)] = sig
                return carry

            for g in range(PASS // LANES):
                group(g, 0)

        for p in range(NPASS):
            do_pass(p)

        pltpu.sync_copy(outv, out_hbm.at[pl.ds(base, BPW)])

    return cmf_kernel


_cmf = _make_kernel()


def kernel(users, items, user_emb, item_emb):
    uemb2 = user_emb.reshape(SROWS, 128)
    iemb2 = item_emb.reshape(SROWS, 128)
    return _cmf(users, items, uemb2, iemb2)
